# speedup vs baseline: 1.0000x; 1.0000x over previous
import jax
import jax.numpy as jnp
from jax.experimental import pallas as pl

N = 100000
HEADS = 4
NUM_GRAPHS = 128


def _gat(x, W, a_s, a_d, b, src, dst):
    co = W.shape[1] // HEADS
    h = (x @ W).reshape(N, HEADS, co)
    es = jnp.sum(h * a_s[None], axis=-1)
    ed = jnp.sum(h * a_d[None], axis=-1)
    e = jax.nn.leaky_relu(es[src] + ed[dst], 0.2)
    m = jax.lax.stop_gradient(jax.ops.segment_max(e, dst, num_segments=N))
    m = jnp.where(jnp.isfinite(m), m, 0.0)
    ex = jnp.exp(e - m[dst])
    den = jax.ops.segment_sum(ex, dst, num_segments=N)
    alpha = ex / (den[dst] + 1e-16)
    out = jax.ops.segment_sum(h[src] * alpha[:, :, None], dst, num_segments=N)
    return out.mean(axis=1) + b


def _bn(x, g, be):
    return x / jnp.sqrt(1.0 + 1e-5) * g + be


def _final_proj_kernel(pooled_ref, w_ref, b_ref, o_ref):
    o_ref[...] = pooled_ref[...] @ w_ref[...] + b_ref[0, 0]


def kernel(x, edge_index, batch, W0, as0, ad0, b0, g0, be0, W1, as1, ad1, b1, g1, be1, W2, as2, ad2, b2, g2, be2, W3, as3, ad3, b3, g3, be3, Wout, bout):
    loop = jnp.arange(N, dtype=edge_index.dtype)
    src = jnp.concatenate([edge_index[0], loop])
    dst = jnp.concatenate([edge_index[1], loop])

    def block(h, W, a_s, a_d, b, g, be):
        return jax.nn.elu(_bn(_gat(h, W, a_s, a_d, b, src, dst), g, be))

    h = block(x, W0, as0, ad0, b0, g0, be0)
    h = block(h, W1, as1, ad1, b1, g1, be1)
    h = block(h, W2, as2, ad2, b2, g2, be2)
    h = block(h, W3, as3, ad3, b3, g3, be3)
    sums = jax.ops.segment_sum(h, batch, num_segments=NUM_GRAPHS)
    cnt = jax.ops.segment_sum(jnp.ones((N, 1), h.dtype), batch, num_segments=NUM_GRAPHS)
    pooled = sums / jnp.maximum(cnt, 1.0)
    out = pl.pallas_call(
        _final_proj_kernel,
        out_shape=jax.ShapeDtypeStruct((NUM_GRAPHS, 1), jnp.float32),
    )(pooled, Wout, bout.reshape(1, 1))
    return out


# speedup vs baseline: 29.3773x; 29.3766x over previous
import functools
import jax
import jax.numpy as jnp
import numpy as np
from jax import lax
from jax.experimental import pallas as pl
from jax.experimental.pallas import tpu as pltpu, tpu_sc as plsc

N = 100000
E = 1600000
HEADS = 4
NUM_GRAPHS = 128
INS = [9, 64, 32, 16]
OUTS = [64, 32, 16, 8]

NW = 32
TE = 128
Cn = 200
NCH = 512
Npad = NCH * Cn
CPT = NCH // NW
BN = 512
NB = Npad // BN
BND_LEN = 544
BNS = 1.0 / np.sqrt(1.0 + 1e-5)


def _dense_kernel(x_ref, w_ref, a_ref, h_ref, es_ref, edm_ref):
    h = jnp.dot(x_ref[...], w_ref[...], preferred_element_type=jnp.float32)
    esed = jnp.dot(h, a_ref[...], preferred_element_type=jnp.float32)
    es = esed[:, 0:4]
    ed = esed[:, 4:8]
    t = es + ed
    m = jnp.maximum(t, 0.2 * t)
    h_ref[...] = h
    es_ref[...] = es
    edm_ref[...] = jnp.concatenate([ed, m], axis=1)


def _epilogue(num, den, aux, co):
    parts = []
    for hd in range(HEADS):
        parts.append(num[:, hd * co:(hd + 1) * co] / (den[:, hd:hd + 1] + 1e-16))
    x = (parts[0] + parts[1] + parts[2] + parts[3]) * 0.25
    b = aux[0:1, 0:co]
    g = aux[1:2, 0:co]
    be = aux[2:3, 0:co]
    y = (x + b) * (BNS * g) + be
    return jnp.where(y > 0, y, jnp.exp(jnp.minimum(y, 0.0)) - 1.0)


def _mid_kernel(co_prev, num_ref, den_ref, aux_ref, w_ref, a_ref,
                h_ref, es_ref, edm_ref):
    x = _epilogue(num_ref[...], den_ref[...], aux_ref[...], co_prev)
    h = jnp.dot(x, w_ref[...], preferred_element_type=jnp.float32)
    esed = jnp.dot(h, a_ref[...], preferred_element_type=jnp.float32)
    es = esed[:, 0:4]
    ed = esed[:, 4:8]
    t = es + ed
    m = jnp.maximum(t, 0.2 * t)
    h_ref[...] = h
    es_ref[...] = es
    edm_ref[...] = jnp.concatenate([ed, m], axis=1)


def _final_kernel(co_prev, num_ref, den_ref, aux_ref, batch_ref, waug_ref,
                  out_ref, acc_ref):
    i = pl.program_id(0)

    @pl.when(i == 0)
    def _():
        acc_ref[...] = jnp.zeros_like(acc_ref)

    hb = _epilogue(num_ref[...], den_ref[...], aux_ref[...], co_prev)
    bvec = batch_ref[0, 0, :]
    onehot = (bvec[:, None] == lax.broadcasted_iota(jnp.int32, (BN, NUM_GRAPHS), 1)
              ).astype(jnp.float32)
    hbaug = jnp.concatenate(
        [hb, jnp.ones((BN, 1), jnp.float32), jnp.zeros((BN, 7), jnp.float32)],
        axis=1)
    acc_ref[...] += jnp.dot(onehot.T, hbaug, preferred_element_type=jnp.float32)

    @pl.when(i == NB - 1)
    def _():
        acc = acc_ref[...]
        pooled = acc[:, 0:8] / jnp.maximum(acc[:, 8:9], 1.0)
        paug = jnp.concatenate(
            [pooled, jnp.ones((NUM_GRAPHS, 1), jnp.float32),
             jnp.zeros((NUM_GRAPHS, 7), jnp.float32)], axis=1)
        out_ref[...] = jnp.dot(paug, waug_ref[...],
                               preferred_element_type=jnp.float32)


def _make_sc_edge_kernel(F):
    co = F // HEADS
    P = max(1, 128 // F)
    G = max(1, F // 128)
    NQ = F // 16
    scmesh = plsc.VectorSubcoreMesh(core_axis_name="c", subcore_axis_name="s")

    def body(htab, estab, edtab, srcs, dsts, bnds, num_hbm, den_hbm,
             src_v, dst_v, sidx_v, esi_v, esb_v, edb_v, exb_v, bnd_v,
             den_v, num_v, hbufA, hbufB, sem):
        wid = lax.axis_index("s") * 2 + lax.axis_index("c")
        pltpu.sync_copy(bnds, bnd_v)

        lane = lax.iota(jnp.int32, 16)
        lane4 = lane % 4
        msk4 = lane < 4
        rep4 = lane // 4
        permm = 4 + lane4
        perms = [(lane + q * 16) // co for q in range(NQ)]

        def ccbody(cc, _):
            c = cc * NW + wid
            e0 = bnd_v[pl.ds(c, 16)][0]
            e1 = bnd_v[pl.ds(c + 1, 16)][0]
            a0 = (e0 // 8) * 8
            nt = (e1 - a0 + (TE - 1)) // TE

            pltpu.sync_copy(edtab.at[pl.ds(c * Cn * 8, Cn * 8)],
                            edb_v.at[pl.ds(0, Cn * 8)])

            def zb(i, _):
                num_v[pl.ds(i * 16, 16)] = jnp.zeros((16,), jnp.float32)
                return 0
            lax.fori_loop(0, (Cn * F) // 16, zb, 0)

            def zb2(i, _):
                den_v[pl.ds(i * 16, 16)] = jnp.zeros((16,), jnp.float32)
                return 0
            lax.fori_loop(0, (Cn * 4) // 16, zb2, 0)

            def tbody(t, _):
                eb = a0 + t * TE
                pltpu.sync_copy(srcs.at[pl.ds(eb, TE)], src_v.at[pl.ds(0, TE)])
                pltpu.sync_copy(dsts.at[pl.ds(eb, TE)], dst_v.at[pl.ds(0, TE)])

                def ib(k, _):
                    s16 = src_v[pl.ds(k * 4, 16)]
                    rep = s16.at[rep4].get(mode="promise_in_bounds")
                    esi_v[pl.ds(k * 16, 16)] = rep * 4 + lane4
                    return 0
                lax.fori_loop(0, TE // 4, ib, 0)

                if G == 2:
                    def gb(k, _):
                        s16 = src_v[pl.ds(k * 16, 16)]
                        sidx_v[pl.ds(k * 16, 16)] = s16 * 2
                        sidx_v[pl.ds(TE + k * 16, 16)] = s16 * 2 + 1
                        return 0
                    lax.fori_loop(0, TE // 16, gb, 0)
                    cpA = pltpu.async_copy(
                        htab.at[sidx_v.at[pl.ds(0, TE)]], hbufA, sem)
                    cpA.wait()
                    cpB = pltpu.async_copy(
                        htab.at[sidx_v.at[pl.ds(TE, TE)]], hbufB, sem)
                    cpB.wait()
                elif P > 1:
                    def gb(k, _):
                        s16 = src_v[pl.ds(k * 16, 16)]
                        sidx_v[pl.ds(k * 16, 16)] = s16 // P
                        return 0
                    lax.fori_loop(0, TE // 16, gb, 0)
                    pltpu.async_copy(
                        htab.at[sidx_v.at[pl.ds(0, TE)]], hbufA, sem).wait()
                else:
                    pltpu.async_copy(
                        htab.at[src_v.at[pl.ds(0, TE)]], hbufA, sem).wait()

                for g in range(4):
                    pltpu.async_copy(
                        estab.at[esi_v.at[pl.ds(g * 128, 128)]],
                        esb_v.at[pl.ds(g * 128, 128)], sem).wait()

                lo = jnp.maximum(e0 - eb, 0)
                hi = jnp.minimum(e1 - eb, TE)

                def ebody(e, _):
                    dl = dst_v[pl.ds(e, 16)][0] - c * Cn
                    a = esb_v[pl.ds(e * 4, 16)]
                    bv = edb_v[pl.ds(dl * 8, 16)]
                    tt = a + bv
                    u = jnp.maximum(tt, 0.2 * tt)
                    m16 = bv.at[permm].get(mode="promise_in_bounds")
                    w = jnp.exp(u - m16)
                    exb_v[pl.ds(e * 16, 16)] = w
                    plsc.addupdate_scatter(den_v, [dl * 4 + lane4], w, mask=msk4)
                    return 0
                lax.fori_loop(lo, hi, ebody, 0)

                def abody(e, _):
                    dl = dst_v[pl.ds(e, 16)][0] - c * Cn
                    wv = exb_v[pl.ds(e * 16, 16)]
                    nbase = dl * F
                    if P > 1:
                        off = (src_v[pl.ds(e, 16)][0] % P) * F
                    else:
                        off = 0
                    for q in range(NQ):
                        sc = wv.at[perms[q]].get(mode="promise_in_bounds")
                        if G == 2 and q >= 8:
                            hrow = hbufB[e, pl.ds((q - 8) * 16, 16)]
                        else:
                            hrow = hbufA[e, pl.ds(off + q * 16, 16)]
                        num_v[pl.ds(nbase + q * 16, 16)] = (
                            num_v[pl.ds(nbase + q * 16, 16)] + hrow * sc)
                    return 0
                lax.fori_loop(lo, hi, abody, 0)
                return 0
            lax.fori_loop(0, nt, tbody, 0)

            pltpu.sync_copy(num_v, num_hbm.at[pl.ds(c * Cn * F, Cn * F)])
            pltpu.sync_copy(den_v.at[pl.ds(0, Cn * 4)],
                            den_hbm.at[pl.ds(c * Cn * 4, Cn * 4)])
            return 0
        lax.fori_loop(0, CPT, ccbody, 0)

    nrows = (Npad * F) // 128
    hbufB_shape = (TE, 128) if F == 256 else (8, 128)
    return functools.partial(
        pl.kernel, mesh=scmesh,
        out_type=(jax.ShapeDtypeStruct((Npad * F,), jnp.float32),
                  jax.ShapeDtypeStruct((Npad * 4,), jnp.float32)),
        scratch_types=[
            pltpu.VMEM((TE + 16,), jnp.int32),
            pltpu.VMEM((TE + 16,), jnp.int32),
            pltpu.VMEM((2 * TE,), jnp.int32),
            pltpu.VMEM((TE * 4,), jnp.int32),
            pltpu.VMEM((TE * 4 + 16,), jnp.float32),
            pltpu.VMEM((Cn * 8 + 16,), jnp.float32),
            pltpu.VMEM((TE * 16 + 16,), jnp.float32),
            pltpu.VMEM((BND_LEN,), jnp.int32),
            pltpu.VMEM((Cn * 4 + 16,), jnp.float32),
            pltpu.VMEM((Cn * F,), jnp.float32),
            pltpu.VMEM((TE, 128), jnp.float32),
            pltpu.VMEM(hbufB_shape, jnp.float32),
            pltpu.SemaphoreType.DMA,
        ],
        compiler_params=pltpu.CompilerParams(needs_layout_passes=False),
    )(body)


def _make_A(a_s, a_d, co):
    F = HEADS * co
    A = jnp.zeros((F, 128), jnp.float32)
    for hd in range(HEADS):
        A = A.at[hd * co:(hd + 1) * co, hd].set(a_s[hd])
        A = A.at[hd * co:(hd + 1) * co, 4 + hd].set(a_d[hd])
    return A


def _make_aux(b, g, be):
    co = b.shape[0]
    aux = jnp.zeros((8, 128), jnp.float32)
    aux = aux.at[0, :co].set(b)
    aux = aux.at[1, :co].set(g)
    aux = aux.at[2, :co].set(be)
    return aux


def kernel(x, edge_index, batch, W0, as0, ad0, b0, g0, be0, W1, as1, ad1, b1,
           g1, be1, W2, as2, ad2, b2, g2, be2, W3, as3, ad3, b3, g3, be3,
           Wout, bout):
    loop = jnp.arange(N, dtype=jnp.int32)
    src_all = jnp.concatenate([edge_index[0], loop])
    dst_all = jnp.concatenate([edge_index[1], loop])
    order = jnp.argsort(dst_all)
    ssrc = src_all[order]
    sdst = dst_all[order]
    ET = E + N
    Elen = ((ET + TE - 1) // TE) * TE + TE
    ssrc = jnp.concatenate(
        [ssrc, jnp.zeros((Elen - ET,), jnp.int32)])
    sdst = jnp.concatenate(
        [sdst, jnp.full((Elen - ET,), Npad, jnp.int32)])
    bounds = jnp.searchsorted(sdst, jnp.arange(NCH + 1, dtype=jnp.int32) * Cn
                              ).astype(jnp.int32)
    bounds = jnp.concatenate(
        [bounds, jnp.full((BND_LEN - NCH - 1,), ET, jnp.int32)])

    xpad = jnp.zeros((Npad, 16), jnp.float32).at[:N, :9].set(x)
    W0p = jnp.zeros((16, 256), jnp.float32).at[:9, :].set(W0)

    weights = [(W0p, _make_A(as0, ad0, 64), None),
               (W1, _make_A(as1, ad1, 32), _make_aux(b0, g0, be0)),
               (W2, _make_A(as2, ad2, 16), _make_aux(b1, g1, be1)),
               (W3, _make_A(as3, ad3, 8), _make_aux(b2, g2, be2))]

    F0 = 256
    h, es, edm = pl.pallas_call(
        _dense_kernel,
        grid=(NB,),
        in_specs=[pl.BlockSpec((BN, 16), lambda i: (i, 0)),
                  pl.BlockSpec((16, F0), lambda i: (0, 0)),
                  pl.BlockSpec((F0, 128), lambda i: (0, 0))],
        out_specs=[pl.BlockSpec((BN, F0), lambda i: (i, 0)),
                   pl.BlockSpec((BN, 4), lambda i: (i, 0)),
                   pl.BlockSpec((BN, 8), lambda i: (i, 0))],
        out_shape=[jax.ShapeDtypeStruct((Npad, F0), jnp.float32),
                   jax.ShapeDtypeStruct((Npad, 4), jnp.float32),
                   jax.ShapeDtypeStruct((Npad, 8), jnp.float32)],
    )(xpad, weights[0][0], weights[0][1])

    num = den = None
    for li in range(4):
        co = OUTS[li]
        F = HEADS * co
        if li > 0:
            co_prev = OUTS[li - 1]
            Fp = HEADS * co_prev
            Wl, Al, auxl = weights[li]
            ci = INS[li]
            h, es, edm = pl.pallas_call(
                functools.partial(_mid_kernel, co_prev),
                grid=(NB,),
                in_specs=[pl.BlockSpec((BN, Fp), lambda i: (i, 0)),
                          pl.BlockSpec((BN, 4), lambda i: (i, 0)),
                          pl.BlockSpec((8, 128), lambda i: (0, 0)),
                          pl.BlockSpec((ci, F), lambda i: (0, 0)),
                          pl.BlockSpec((F, 128), lambda i: (0, 0))],
                out_specs=[pl.BlockSpec((BN, F), lambda i: (i, 0)),
                           pl.BlockSpec((BN, 4), lambda i: (i, 0)),
                           pl.BlockSpec((BN, 8), lambda i: (i, 0))],
                out_shape=[jax.ShapeDtypeStruct((Npad, F), jnp.float32),
                           jax.ShapeDtypeStruct((Npad, 4), jnp.float32),
                           jax.ShapeDtypeStruct((Npad, 8), jnp.float32)],
            )(num, den, auxl, Wl, Al)

        htab = h.reshape(((Npad * F) // 128, 128))
        estab = es.reshape((Npad * 4,))
        edtab = edm.reshape((Npad * 8,))
        numf, denf = _make_sc_edge_kernel(F)(
            htab, estab, edtab, ssrc, sdst, bounds)
        num = numf.reshape((Npad, F))
        den = denf.reshape((Npad, 4))

    batchp = jnp.concatenate(
        [batch.astype(jnp.int32), jnp.full((Npad - N,), NUM_GRAPHS, jnp.int32)]
    ).reshape((NB, 1, BN))
    aux3 = _make_aux(b3, g3, be3)
    waug = jnp.zeros((16, 128), jnp.float32)
    waug = waug.at[0:8, 0].set(Wout[:, 0])
    waug = waug.at[8, 0].set(bout[0])

    out = pl.pallas_call(
        functools.partial(_final_kernel, OUTS[3]),
        grid=(NB,),
        in_specs=[pl.BlockSpec((BN, 32), lambda i: (i, 0)),
                  pl.BlockSpec((BN, 4), lambda i: (i, 0)),
                  pl.BlockSpec((8, 128), lambda i: (0, 0)),
                  pl.BlockSpec((1, 1, BN), lambda i: (i, 0, 0)),
                  pl.BlockSpec((16, 128), lambda i: (0, 0))],
        out_specs=pl.BlockSpec((NUM_GRAPHS, 128), lambda i: (0, 0)),
        out_shape=jax.ShapeDtypeStruct((NUM_GRAPHS, 128), jnp.float32),
        scratch_shapes=[pltpu.VMEM((NUM_GRAPHS, 16), jnp.float32)],
    )(num, den, aux3, batchp, waug)
    return out[:, 0:1]


# speedup vs baseline: 33.8835x; 1.1534x over previous
import functools
import jax
import jax.numpy as jnp
import numpy as np
from jax import lax
from jax.experimental import pallas as pl
from jax.experimental.pallas import tpu as pltpu, tpu_sc as plsc

N = 100000
E = 1600000
HEADS = 4
NUM_GRAPHS = 128
INS = [9, 64, 32, 16]
OUTS = [64, 32, 16, 8]

NW = 32
TE = 128
Cn = 200
NCH = 512
Npad = NCH * Cn
CPT = NCH // NW
BN = 512
NB = Npad // BN
BND_LEN = 544
BNS = 1.0 / np.sqrt(1.0 + 1e-5)


def _dense_kernel(x_ref, w_ref, a_ref, h_ref, es_ref, edm_ref):
    h = jnp.dot(x_ref[...], w_ref[...], preferred_element_type=jnp.float32, precision=lax.Precision.HIGHEST)
    esed = jnp.dot(h, a_ref[...], preferred_element_type=jnp.float32, precision=lax.Precision.HIGHEST)
    es = esed[:, 0:4]
    ed = esed[:, 4:8]
    t = es + ed
    m = jnp.maximum(t, 0.2 * t)
    h_ref[...] = h
    es_ref[...] = es
    edm_ref[...] = jnp.concatenate([ed, m], axis=1)


def _epilogue(num, den, aux, co):
    parts = []
    for hd in range(HEADS):
        parts.append(num[:, hd * co:(hd + 1) * co] / (den[:, hd:hd + 1] + 1e-16))
    x = (parts[0] + parts[1] + parts[2] + parts[3]) * 0.25
    b = aux[0:1, 0:co]
    g = aux[1:2, 0:co]
    be = aux[2:3, 0:co]
    y = (x + b) * (BNS * g) + be
    yn = jnp.minimum(y, 0.0)
    small = yn * (1.0 + yn * (0.5 + yn * (1.0 / 6.0 + yn * (1.0 / 24.0))))
    em1 = jnp.where(yn > -0.1, small, jnp.exp(yn) - 1.0)
    return jnp.where(y > 0, y, em1)


def _mid_kernel(co_prev, num_ref, den_ref, aux_ref, w_ref, a_ref,
                h_ref, es_ref, edm_ref):
    x = _epilogue(num_ref[...], den_ref[...], aux_ref[...], co_prev)
    h = jnp.dot(x, w_ref[...], preferred_element_type=jnp.float32, precision=lax.Precision.HIGHEST)
    esed = jnp.dot(h, a_ref[...], preferred_element_type=jnp.float32, precision=lax.Precision.HIGHEST)
    es = esed[:, 0:4]
    ed = esed[:, 4:8]
    t = es + ed
    m = jnp.maximum(t, 0.2 * t)
    h_ref[...] = h
    es_ref[...] = es
    edm_ref[...] = jnp.concatenate([ed, m], axis=1)


def _final_kernel(co_prev, num_ref, den_ref, aux_ref, batch_ref, waug_ref,
                  out_ref, acc_ref):
    i = pl.program_id(0)

    @pl.when(i == 0)
    def _():
        acc_ref[...] = jnp.zeros_like(acc_ref)

    hb = _epilogue(num_ref[...], den_ref[...], aux_ref[...], co_prev)
    bvec = batch_ref[0, 0, :]
    onehot = (bvec[:, None] == lax.broadcasted_iota(jnp.int32, (BN, NUM_GRAPHS), 1)
              ).astype(jnp.float32)
    hbaug = jnp.concatenate(
        [hb, jnp.ones((BN, 1), jnp.float32), jnp.zeros((BN, 7), jnp.float32)],
        axis=1)
    acc_ref[...] += jnp.dot(onehot.T, hbaug, preferred_element_type=jnp.float32, precision=lax.Precision.HIGHEST)

    @pl.when(i == NB - 1)
    def _():
        acc = acc_ref[...]
        pooled = acc[:, 0:8] / jnp.maximum(acc[:, 8:9], 1.0)
        paug = jnp.concatenate(
            [pooled, jnp.ones((NUM_GRAPHS, 1), jnp.float32),
             jnp.zeros((NUM_GRAPHS, 7), jnp.float32)], axis=1)
        out_ref[...] = jnp.dot(paug, waug_ref[...],
                               preferred_element_type=jnp.float32, precision=lax.Precision.HIGHEST)


def _make_sc_edge_kernel(F):
    co = F // HEADS
    P = max(1, 128 // F)
    G = max(1, F // 128)
    NQ = F // 16
    scmesh = plsc.VectorSubcoreMesh(core_axis_name="c", subcore_axis_name="s")

    def body(htab, estab, edtab, srcs, dsts, bnds, num_hbm, den_hbm,
             src_v, dst_v, sidx_v, esi_v, esb_v, edb_v, exb_v, bnd_v,
             den_v, num_v, hbufA, hbufB, sem):
        wid = lax.axis_index("s") * 2 + lax.axis_index("c")
        pltpu.sync_copy(bnds, bnd_v)

        lane = lax.iota(jnp.int32, 16)
        lane4 = lane % 4
        msk4 = lane < 4
        rep4 = lane // 4
        permm = 4 + lane4
        perms = [(lane + q * 16) // co for q in range(NQ)]

        def ccbody(cc, _):
            c = cc * NW + wid
            e0 = bnd_v[pl.ds(c, 16)][0]
            e1 = bnd_v[pl.ds(c + 1, 16)][0]
            a0 = (e0 // 8) * 8
            nt = (e1 - a0 + (TE - 1)) // TE

            pltpu.sync_copy(edtab.at[pl.ds(c * Cn * 8, Cn * 8)],
                            edb_v.at[pl.ds(0, Cn * 8)])

            def zb(i, _):
                num_v[pl.ds(i * 16, 16)] = jnp.zeros((16,), jnp.float32)
                return 0
            lax.fori_loop(0, (Cn * F) // 16, zb, 0)

            def zb2(i, _):
                den_v[pl.ds(i * 16, 16)] = jnp.zeros((16,), jnp.float32)
                return 0
            lax.fori_loop(0, (Cn * 4) // 16, zb2, 0)

            def tbody(t, _):
                eb = a0 + t * TE
                cp1 = pltpu.async_copy(srcs.at[pl.ds(eb, TE)],
                                       src_v.at[pl.ds(0, TE)], sem)
                cp2 = pltpu.async_copy(dsts.at[pl.ds(eb, TE)],
                                       dst_v.at[pl.ds(0, TE)], sem)
                cp1.wait()
                cp2.wait()

                def ib(k, _):
                    s16 = src_v[pl.ds(k * 4, 16)]
                    rep = s16.at[rep4].get(mode="promise_in_bounds")
                    esi_v[pl.ds(k * 16, 16)] = rep * 4 + lane4
                    return 0
                lax.fori_loop(0, TE // 4, ib, 0)

                cps = []
                if G == 2:
                    def gb(k, _):
                        s16 = src_v[pl.ds(k * 16, 16)]
                        sidx_v[pl.ds(k * 16, 16)] = s16 * 2
                        sidx_v[pl.ds(TE + k * 16, 16)] = s16 * 2 + 1
                        return 0
                    lax.fori_loop(0, TE // 16, gb, 0)
                    cps.append(pltpu.async_copy(
                        htab.at[sidx_v.at[pl.ds(0, TE)]], hbufA, sem))
                    cps.append(pltpu.async_copy(
                        htab.at[sidx_v.at[pl.ds(TE, TE)]], hbufB, sem))
                elif P > 1:
                    def gb(k, _):
                        s16 = src_v[pl.ds(k * 16, 16)]
                        sidx_v[pl.ds(k * 16, 16)] = s16 // P
                        return 0
                    lax.fori_loop(0, TE // 16, gb, 0)
                    cps.append(pltpu.async_copy(
                        htab.at[sidx_v.at[pl.ds(0, TE)]], hbufA, sem))
                else:
                    cps.append(pltpu.async_copy(
                        htab.at[src_v.at[pl.ds(0, TE)]], hbufA, sem))

                for g in range(4):
                    cps.append(pltpu.async_copy(
                        estab.at[esi_v.at[pl.ds(g * 128, 128)]],
                        esb_v.at[pl.ds(g * 128, 128)], sem))
                for cp in cps:
                    cp.wait()

                lo = jnp.maximum(e0 - eb, 0)
                hi = jnp.minimum(e1 - eb, TE)

                def ebody(e, _):
                    dl = dst_v[pl.ds(e, 16)][0] - c * Cn
                    a = esb_v[pl.ds(e * 4, 16)]
                    bv = edb_v[pl.ds(dl * 8, 16)]
                    tt = a + bv
                    u = jnp.maximum(tt, 0.2 * tt)
                    m16 = bv.at[permm].get(mode="promise_in_bounds")
                    w = jnp.exp(u - m16)
                    exb_v[pl.ds(e * 16, 16)] = w
                    plsc.addupdate_scatter(den_v, [dl * 4 + lane4], w, mask=msk4)
                    return 0
                lax.fori_loop(lo, hi, ebody, 0)

                def abody(e, _):
                    dl = dst_v[pl.ds(e, 16)][0] - c * Cn
                    wv = exb_v[pl.ds(e * 16, 16)]
                    nbase = dl * F
                    if P > 1:
                        off = (src_v[pl.ds(e, 16)][0] % P) * F
                    else:
                        off = 0
                    for q in range(NQ):
                        sc = wv.at[perms[q]].get(mode="promise_in_bounds")
                        if G == 2 and q >= 8:
                            hrow = hbufB[e, pl.ds((q - 8) * 16, 16)]
                        else:
                            hrow = hbufA[e, pl.ds(off + q * 16, 16)]
                        num_v[pl.ds(nbase + q * 16, 16)] = (
                            num_v[pl.ds(nbase + q * 16, 16)] + hrow * sc)
                    return 0
                lax.fori_loop(lo, hi, abody, 0)
                return 0
            lax.fori_loop(0, nt, tbody, 0)

            pltpu.sync_copy(num_v, num_hbm.at[pl.ds(c * Cn * F, Cn * F)])
            pltpu.sync_copy(den_v.at[pl.ds(0, Cn * 4)],
                            den_hbm.at[pl.ds(c * Cn * 4, Cn * 4)])
            return 0
        lax.fori_loop(0, CPT, ccbody, 0)

    nrows = (Npad * F) // 128
    hbufB_shape = (TE, 128) if F == 256 else (8, 128)
    return functools.partial(
        pl.kernel, mesh=scmesh,
        out_type=(jax.ShapeDtypeStruct((Npad * F,), jnp.float32),
                  jax.ShapeDtypeStruct((Npad * 4,), jnp.float32)),
        scratch_types=[
            pltpu.VMEM((TE + 16,), jnp.int32),
            pltpu.VMEM((TE + 16,), jnp.int32),
            pltpu.VMEM((2 * TE,), jnp.int32),
            pltpu.VMEM((TE * 4,), jnp.int32),
            pltpu.VMEM((TE * 4 + 16,), jnp.float32),
            pltpu.VMEM((Cn * 8 + 16,), jnp.float32),
            pltpu.VMEM((TE * 16 + 16,), jnp.float32),
            pltpu.VMEM((BND_LEN,), jnp.int32),
            pltpu.VMEM((Cn * 4 + 16,), jnp.float32),
            pltpu.VMEM((Cn * F,), jnp.float32),
            pltpu.VMEM((TE, 128), jnp.float32),
            pltpu.VMEM(hbufB_shape, jnp.float32),
            pltpu.SemaphoreType.DMA,
        ],
        compiler_params=pltpu.CompilerParams(needs_layout_passes=False),
    )(body)


def _make_A(a_s, a_d, co):
    F = HEADS * co
    A = jnp.zeros((F, 128), jnp.float32)
    for hd in range(HEADS):
        A = A.at[hd * co:(hd + 1) * co, hd].set(a_s[hd])
        A = A.at[hd * co:(hd + 1) * co, 4 + hd].set(a_d[hd])
    return A


def _make_aux(b, g, be):
    co = b.shape[0]
    aux = jnp.zeros((8, 128), jnp.float32)
    aux = aux.at[0, :co].set(b)
    aux = aux.at[1, :co].set(g)
    aux = aux.at[2, :co].set(be)
    return aux


def kernel(x, edge_index, batch, W0, as0, ad0, b0, g0, be0, W1, as1, ad1, b1,
           g1, be1, W2, as2, ad2, b2, g2, be2, W3, as3, ad3, b3, g3, be3,
           Wout, bout):
    loop = jnp.arange(N, dtype=jnp.int32)
    src_all = jnp.concatenate([edge_index[0], loop])
    dst_all = jnp.concatenate([edge_index[1], loop])
    order = jnp.argsort(dst_all)
    ssrc = src_all[order]
    sdst = dst_all[order]
    ET = E + N
    Elen = ((ET + TE - 1) // TE) * TE + TE
    ssrc = jnp.concatenate(
        [ssrc, jnp.zeros((Elen - ET,), jnp.int32)])
    sdst = jnp.concatenate(
        [sdst, jnp.full((Elen - ET,), Npad, jnp.int32)])
    bounds = jnp.searchsorted(sdst, jnp.arange(NCH + 1, dtype=jnp.int32) * Cn
                              ).astype(jnp.int32)
    bounds = jnp.concatenate(
        [bounds, jnp.full((BND_LEN - NCH - 1,), ET, jnp.int32)])

    xpad = jnp.zeros((Npad, 16), jnp.float32).at[:N, :9].set(x)
    W0p = jnp.zeros((16, 256), jnp.float32).at[:9, :].set(W0)

    weights = [(W0p, _make_A(as0, ad0, 64), None),
               (W1, _make_A(as1, ad1, 32), _make_aux(b0, g0, be0)),
               (W2, _make_A(as2, ad2, 16), _make_aux(b1, g1, be1)),
               (W3, _make_A(as3, ad3, 8), _make_aux(b2, g2, be2))]

    F0 = 256
    h, es, edm = pl.pallas_call(
        _dense_kernel,
        grid=(NB,),
        in_specs=[pl.BlockSpec((BN, 16), lambda i: (i, 0)),
                  pl.BlockSpec((16, F0), lambda i: (0, 0)),
                  pl.BlockSpec((F0, 128), lambda i: (0, 0))],
        out_specs=[pl.BlockSpec((BN, F0), lambda i: (i, 0)),
                   pl.BlockSpec((BN, 4), lambda i: (i, 0)),
                   pl.BlockSpec((BN, 8), lambda i: (i, 0))],
        out_shape=[jax.ShapeDtypeStruct((Npad, F0), jnp.float32),
                   jax.ShapeDtypeStruct((Npad, 4), jnp.float32),
                   jax.ShapeDtypeStruct((Npad, 8), jnp.float32)],
    )(xpad, weights[0][0], weights[0][1])

    num = den = None
    for li in range(4):
        co = OUTS[li]
        F = HEADS * co
        if li > 0:
            co_prev = OUTS[li - 1]
            Fp = HEADS * co_prev
            Wl, Al, auxl = weights[li]
            ci = INS[li]
            h, es, edm = pl.pallas_call(
                functools.partial(_mid_kernel, co_prev),
                grid=(NB,),
                in_specs=[pl.BlockSpec((BN, Fp), lambda i: (i, 0)),
                          pl.BlockSpec((BN, 4), lambda i: (i, 0)),
                          pl.BlockSpec((8, 128), lambda i: (0, 0)),
                          pl.BlockSpec((ci, F), lambda i: (0, 0)),
                          pl.BlockSpec((F, 128), lambda i: (0, 0))],
                out_specs=[pl.BlockSpec((BN, F), lambda i: (i, 0)),
                           pl.BlockSpec((BN, 4), lambda i: (i, 0)),
                           pl.BlockSpec((BN, 8), lambda i: (i, 0))],
                out_shape=[jax.ShapeDtypeStruct((Npad, F), jnp.float32),
                           jax.ShapeDtypeStruct((Npad, 4), jnp.float32),
                           jax.ShapeDtypeStruct((Npad, 8), jnp.float32)],
            )(num, den, auxl, Wl, Al)

        htab = h.reshape(((Npad * F) // 128, 128))
        estab = es.reshape((Npad * 4,))
        edtab = edm.reshape((Npad * 8,))
        numf, denf = _make_sc_edge_kernel(F)(
            htab, estab, edtab, ssrc, sdst, bounds)
        num = numf.reshape((Npad, F))
        den = denf.reshape((Npad, 4))

    batchp = jnp.concatenate(
        [batch.astype(jnp.int32), jnp.full((Npad - N,), NUM_GRAPHS, jnp.int32)]
    ).reshape((NB, 1, BN))
    aux3 = _make_aux(b3, g3, be3)
    waug = jnp.zeros((16, 128), jnp.float32)
    waug = waug.at[0:8, 0].set(Wout[:, 0])
    waug = waug.at[8, 0].set(bout[0])

    out = pl.pallas_call(
        functools.partial(_final_kernel, OUTS[3]),
        grid=(NB,),
        in_specs=[pl.BlockSpec((BN, 32), lambda i: (i, 0)),
                  pl.BlockSpec((BN, 4), lambda i: (i, 0)),
                  pl.BlockSpec((8, 128), lambda i: (0, 0)),
                  pl.BlockSpec((1, 1, BN), lambda i: (i, 0, 0)),
                  pl.BlockSpec((16, 128), lambda i: (0, 0))],
        out_specs=pl.BlockSpec((NUM_GRAPHS, 128), lambda i: (0, 0)),
        out_shape=jax.ShapeDtypeStruct((NUM_GRAPHS, 128), jnp.float32),
        scratch_shapes=[pltpu.VMEM((NUM_GRAPHS, 16), jnp.float32)],
    )(num, den, aux3, batchp, waug)
    return out[:, 0:1]
